# Initial kernel scaffold; baseline (speedup 1.0000x reference)
import jax
import jax.numpy as jnp
from jax.experimental import pallas as pl


def kernel(x, y, row_gs, col_gs, val_gs, row_sg, col_sg, val_sg, Wg1_self, Wg1_neigh, Ws1_self, Ws1_neigh, Wg2_self, Wg2_neigh, Ws2_self, Ws2_neigh, WQ, bQ, WK, bK, WV, bV, Wpg, bpg, Wps, bps, Wlx, Wly):
    raise NotImplementedError("write your pallas kernel here")


# speedup vs baseline: 2.3076x; 2.3076x over previous
import functools

import jax
import jax.numpy as jnp
from jax import lax
from jax.experimental import pallas as pl
from jax.experimental.pallas import tpu as pltpu
from jax.experimental.pallas import tpu_sc as plsc


def _spmm_sc_call(row, col, val, x, n_out):
    e_total = row.shape[0]
    n_src, d = x.shape
    NC, NSUB = 2, 16
    NW = NC * NSUB
    epw = e_total // NW
    K = 80
    nchunks = epw // K
    assert epw * NW == e_total and nchunks * K == epw and d % 16 == 0
    n_oc = n_out // K
    assert n_oc * K == n_out
    n_oit = -(-n_oc // NSUB)

    mesh = plsc.VectorSubcoreMesh(core_axis_name="c", subcore_axis_name="s")

    @functools.partial(
        pl.kernel,
        mesh=mesh,
        out_type=jax.ShapeDtypeStruct((NC, n_out, d), jnp.float32),
        scratch_types=[
            pltpu.VMEM((K,), jnp.int32),
            pltpu.VMEM((K,), jnp.int32),
            pltpu.VMEM((K + 16,), jnp.float32),
            pltpu.VMEM((K, d), jnp.float32),
            pltpu.VMEM_SHARED((n_out, d), jnp.float32),
            pltpu.SemaphoreType.DMA,
        ],
    )
    def spmm(row_hbm, col_hbm, val_hbm, x_hbm, out_hbm,
             rowv, colv, valv, gbuf, acc, sem):
        c = lax.axis_index("c")
        s = lax.axis_index("s")
        wid = c * NSUB + s
        z = jnp.zeros((16,), jnp.float32)
        for k in range(K):
            for j in range(d // 16):
                gbuf[k, pl.ds(j * 16, 16)] = z
        for i in range(n_oit):
            blk = s + NSUB * i

            @pl.when(blk < n_oc)
            def _():
                pltpu.sync_copy(gbuf, acc.at[pl.ds(blk * K, K)])

        plsc.subcore_barrier()
        base = wid * epw

        def chunk_body(ci, carry):
            off = base + ci * K
            pltpu.sync_copy(row_hbm.at[pl.ds(off, K)], rowv)
            pltpu.sync_copy(col_hbm.at[pl.ds(off, K)], colv)
            pltpu.sync_copy(val_hbm.at[pl.ds(off, K)], valv.at[pl.ds(0, K)])
            pltpu.async_copy(x_hbm.at[colv], gbuf, sem).wait()

            def edge_body(k, carry2):
                vv = valv[pl.ds(k, 16)]
                vb = jnp.full((16,), vv[0], jnp.float32)
                for j in range(d // 16):
                    sl = pl.ds(j * 16, 16)
                    gbuf[k, sl] = gbuf[k, sl] * vb
                return carry2

            lax.fori_loop(0, K, edge_body, 0, unroll=4)
            pltpu.sync_copy(gbuf, acc.at[rowv], add=True)
            return carry

        lax.fori_loop(0, nchunks, chunk_body, 0)
        plsc.subcore_barrier()
        for i in range(n_oit):
            blk = s + NSUB * i

            @pl.when(blk < n_oc)
            def _():
                pltpu.sync_copy(acc.at[pl.ds(blk * K, K)],
                                out_hbm.at[c, pl.ds(blk * K, K)])

    return spmm(row, col, val, x)


def _dotT(a, w):
    return jax.lax.dot_general(a, w, (((1,), (1,)), ((), ())),
                               preferred_element_type=jnp.float32)


def _sage_layer(x, ngp, y, nsp, wgs, wgn, wss, wsn):
    ng_n, df = x.shape
    ns_n = y.shape[0]
    emb = wgs.shape[0]
    BG = 1000
    grid = ng_n // BG

    def body(x_ref, ng_ref, y_ref, ns_ref, wgs_ref, wgn_ref, wss_ref, wsn_ref,
             g_ref, s_ref):
        i = pl.program_id(0)
        ng = ng_ref[0] + ng_ref[1]
        g_ref[...] = jnp.maximum(
            _dotT(x_ref[...], wgs_ref[...]) + _dotT(ng, wgn_ref[...]), 0.0)

        @pl.when(i == 0)
        def _():
            ns = ns_ref[0] + ns_ref[1]
            s_ref[...] = jnp.maximum(
                _dotT(y_ref[...], wss_ref[...]) + _dotT(ns, wsn_ref[...]), 0.0)

    wspec = pl.BlockSpec((emb, df), lambda i: (0, 0))
    return pl.pallas_call(
        body,
        grid=(grid,),
        in_specs=[
            pl.BlockSpec((BG, df), lambda i: (i, 0)),
            pl.BlockSpec((2, BG, df), lambda i: (0, i, 0)),
            pl.BlockSpec((ns_n, df), lambda i: (0, 0)),
            pl.BlockSpec((2, ns_n, df), lambda i: (0, 0, 0)),
            wspec, wspec, wspec, wspec,
        ],
        out_specs=[
            pl.BlockSpec((BG, emb), lambda i: (i, 0)),
            pl.BlockSpec((ns_n, emb), lambda i: (0, 0)),
        ],
        out_shape=[
            jax.ShapeDtypeStruct((ng_n, emb), jnp.float32),
            jax.ShapeDtypeStruct((ns_n, emb), jnp.float32),
        ],
    )(x, ngp, y, nsp, wgs, wgn, wss, wsn)


def _attn_block(q_src, kv_src, wq, bq, wk, bk, wv, bv, wp, bp, wl):
    q = _dotT(q_src, wq) + bq
    k = _dotT(kv_src, wk) + bk
    v = _dotT(kv_src, wv) + bv
    scores = jnp.dot(q, k.T, preferred_element_type=jnp.float32)
    scores = scores * (1.0 / (q.shape[1] ** 0.5))
    m = jnp.max(scores, axis=-1, keepdims=True)
    e = jnp.exp(scores - m)
    w = e / jnp.sum(e, axis=-1, keepdims=True)
    a = jnp.dot(w, v, preferred_element_type=jnp.float32)
    p = jnp.maximum(_dotT(a, wp) + bp, 0.0)
    dec = _dotT(p, wl)
    nrm = jnp.sqrt(jnp.sum(dec * dec, axis=1, keepdims=True)) + 1e-6
    return dec / nrm


def _attn_s(s2, g2, wq, bq, wk, bk, wv, bv, wps, bps, wly):
    ns_n, emb = s2.shape
    ng_n = g2.shape[0]
    kd = wly.shape[0]
    BS = 200
    grid = ns_n // BS

    def body(s_ref, g_ref, wq_ref, bq_ref, wk_ref, bk_ref, wv_ref, bv_ref,
             wp_ref, bp_ref, wl_ref, yn_ref):
        yn_ref[...] = _attn_block(
            s_ref[...], g_ref[...], wq_ref[...], bq_ref[...], wk_ref[...],
            bk_ref[...], wv_ref[...], bv_ref[...], wp_ref[...], bp_ref[...],
            wl_ref[...])

    wspec = pl.BlockSpec((emb, emb), lambda i: (0, 0))
    bspec = pl.BlockSpec((1, emb), lambda i: (0, 0))
    return pl.pallas_call(
        body,
        grid=(grid,),
        in_specs=[
            pl.BlockSpec((BS, emb), lambda i: (i, 0)),
            pl.BlockSpec((ng_n, emb), lambda i: (0, 0)),
            wspec, bspec, wspec, bspec, wspec, bspec,
            wspec, bspec,
            pl.BlockSpec((kd, emb), lambda i: (0, 0)),
        ],
        out_specs=pl.BlockSpec((BS, kd), lambda i: (i, 0)),
        out_shape=jax.ShapeDtypeStruct((ns_n, kd), jnp.float32),
    )(s2, g2, wq, bq, wk, bk, wv, bv, wps, bps, wly)


def _attn_g_cos(g2, s2, yn, wq, bq, wk, bk, wv, bv, wpg, bpg, wlx):
    ng_n, emb = g2.shape
    ns_n = s2.shape[0]
    kd = wlx.shape[0]
    BG = 1000
    grid = ng_n // BG

    def body(g_ref, s_ref, yn_ref, wq_ref, bq_ref, wk_ref, bk_ref, wv_ref,
             bv_ref, wp_ref, bp_ref, wl_ref, out_ref):
        xn = _attn_block(
            g_ref[...], s_ref[...], wq_ref[...], bq_ref[...], wk_ref[...],
            bk_ref[...], wv_ref[...], bv_ref[...], wp_ref[...], bp_ref[...],
            wl_ref[...])
        cos = jnp.dot(xn, yn_ref[...].T, preferred_element_type=jnp.float32)
        out_ref[...] = (cos + 1.0) * 0.5

    wspec = pl.BlockSpec((emb, emb), lambda i: (0, 0))
    bspec = pl.BlockSpec((1, emb), lambda i: (0, 0))
    return pl.pallas_call(
        body,
        grid=(grid,),
        in_specs=[
            pl.BlockSpec((BG, emb), lambda i: (i, 0)),
            pl.BlockSpec((ns_n, emb), lambda i: (0, 0)),
            pl.BlockSpec((ns_n, kd), lambda i: (0, 0)),
            wspec, bspec, wspec, bspec, wspec, bspec,
            wspec, bspec,
            pl.BlockSpec((kd, emb), lambda i: (0, 0)),
        ],
        out_specs=pl.BlockSpec((BG, ns_n), lambda i: (i, 0)),
        out_shape=jax.ShapeDtypeStruct((ng_n, ns_n), jnp.float32),
    )(g2, s2, yn, wq, bq, wk, bk, wv, bv, wpg, bpg, wlx)


def kernel(x, y, row_gs, col_gs, val_gs, row_sg, col_sg, val_sg,
           Wg1_self, Wg1_neigh, Ws1_self, Ws1_neigh,
           Wg2_self, Wg2_neigh, Ws2_self, Ws2_neigh,
           WQ, bQ, WK, bK, WV, bV, Wpg, bpg, Wps, bps, Wlx, Wly):
    ng_n = x.shape[0]
    ns_n = y.shape[0]
    df = x.shape[1]
    emb = WQ.shape[0]
    bQ2, bK2, bV2 = bQ[None, :], bK[None, :], bV[None, :]
    bpg2, bps2 = bpg[None, :], bps[None, :]

    def _pad_rows(w):
        return jnp.pad(w, ((0, df - w.shape[0]), (0, 0)))

    def _pad_cols(w):
        return jnp.pad(w, ((0, 0), (0, df - w.shape[1])))

    ng1p = _spmm_sc_call(row_gs, col_gs, val_gs, y, ng_n)
    ns1p = _spmm_sc_call(row_sg, col_sg, val_sg, x, ns_n)
    g1, s1 = _sage_layer(x, ng1p, y, ns1p,
                         _pad_rows(Wg1_self), _pad_rows(Wg1_neigh),
                         _pad_rows(Ws1_self), _pad_rows(Ws1_neigh))

    ng2p = _spmm_sc_call(row_gs, col_gs, val_gs, s1, ng_n)
    ns2p = _spmm_sc_call(row_sg, col_sg, val_sg, g1, ns_n)
    g2, s2 = _sage_layer(g1, ng2p, s1, ns2p,
                         _pad_cols(Wg2_self), _pad_cols(Wg2_neigh),
                         _pad_cols(Ws2_self), _pad_cols(Ws2_neigh))

    yn = _attn_s(s2, g2, WQ, bQ2, WK, bK2, WV, bV2, Wps, bps2, Wly)
    return _attn_g_cos(g2, s2, yn, WQ, bQ2, WK, bK2, WV, bV2, Wpg, bpg2, Wlx)
